# baseline (speedup 1.0000x reference)
import jax
import jax.numpy as jnp
from jax.experimental import pallas as pl


def _body(x_ref, w1_ref, w2_ref, w3_ref, b_ref, out_ref):
    out_ref[...] = x_ref[...]


def kernel(x, W1, W2, W3, bias):
    return pl.pallas_call(
        _body,
        out_shape=jax.ShapeDtypeStruct((16,), jnp.float32),
    )(x, W1, W2, W3, bias)
